# baseline (speedup 1.0000x reference)
import functools

import jax
import jax.numpy as jnp
from jax import lax
from jax.experimental import pallas as pl
from jax.experimental.pallas import tpu as pltpu
from jax.experimental.pallas import tpu_sc as plsc

B = 16384
L = 50
LP = 56
DIM = 64
NC = 2
NS = 16
NW = NC * NS
RPW = B // NW
GROUP = 16
NG = RPW // GROUP
CPG = GROUP // 2


def _body(phi2_h, emb_h, bias_h, w_h, b_h, eps_h, h_h,
          idx_v, gbuf, hbuf, dbuf, eps_v, bias_v, w_v, b_v, sem):
    c = lax.axis_index("c")
    s = lax.axis_index("s")
    wid = s * NC + c

    pltpu.sync_copy(bias_h, bias_v)
    pltpu.sync_copy(w_h, w_v)
    pltpu.sync_copy(b_h, b_v)

    def group_body(g, carry):
        base_row = wid * RPW + g * GROUP
        chunk0 = wid * (RPW // 2) + g * CPG

        pltpu.sync_copy(phi2_h.at[pl.ds(chunk0, CPG)], idx_v)
        descs = [
            pltpu.async_copy(
                emb_h.at[idx_v.at[j]], gbuf.at[pl.ds(j * LP * 2, LP * 2)], sem)
            for j in range(CPG)
        ]
        for d in descs:
            d.wait()

        for j in range(CPG):
            for r in range(2):
                rb = j * 2 * LP + r * LP

                def lbody(l, accs, rb=rb):
                    row = rb + l
                    return tuple(accs[i] + gbuf[row, pl.ds(16 * i, 16)]
                                 for i in range(4))

                accs = lax.fori_loop(
                    0, LP, lbody,
                    tuple(jnp.zeros((16,), jnp.float32) for _ in range(4)))

                dot = jnp.zeros((16,), jnp.float32)
                for i in range(4):
                    x = accs[i] + bias_v[pl.ds(16 * i, 16)]
                    t = jnp.exp(-2.0 * jnp.abs(x))
                    th = (1.0 - t) / (1.0 + t)
                    h = jnp.where(x < 0.0, -th, th)
                    hbuf[j * 2 + r, pl.ds(16 * i, 16)] = h
                    dot = dot + h * w_v[pl.ds(16 * i, 16)]
                dbuf[j * 2 + r, :] = dot

        rows = lax.iota(jnp.int32, 16)
        eps16 = jnp.zeros((16,), jnp.float32)
        for k in range(16):
            col = plsc.load_gather(dbuf, [rows, jnp.full((16,), k, jnp.int32)])
            eps16 = eps16 + col

        eps_v[pl.ds(g * GROUP, GROUP)] = eps16 + b_v[...]
        pltpu.sync_copy(hbuf, h_h.at[pl.ds(base_row, GROUP)])
        return carry

    lax.fori_loop(0, NG, group_body, jnp.int32(0))
    pltpu.sync_copy(eps_v, eps_h.at[pl.ds(wid * RPW, RPW)])


@jax.jit
def kernel(phi_a, emb_table, bias, W, b):
    phi_p = jnp.pad(phi_a.astype(jnp.int32), ((0, 0), (0, LP - L)))
    phi2 = phi_p.reshape(B // 2, 2 * LP)
    w1 = W[:, 0]
    b16 = jnp.broadcast_to(b, (16,))

    mesh = plsc.VectorSubcoreMesh(
        core_axis_name="c", subcore_axis_name="s",
        num_cores=NC, num_subcores=NS)
    run = pl.kernel(
        _body,
        out_type=(
            jax.ShapeDtypeStruct((B,), jnp.float32),
            jax.ShapeDtypeStruct((B, DIM), jnp.float32),
        ),
        mesh=mesh,
        compiler_params=pltpu.CompilerParams(
            needs_layout_passes=False, use_tc_tiling_on_sc=False),
        scratch_types=[
            pltpu.VMEM((CPG, 2 * LP), jnp.int32),
            pltpu.VMEM((GROUP * LP, DIM), jnp.float32),
            pltpu.VMEM((GROUP, DIM), jnp.float32),
            pltpu.VMEM((GROUP, 16), jnp.float32),
            pltpu.VMEM((RPW,), jnp.float32),
            pltpu.VMEM((DIM,), jnp.float32),
            pltpu.VMEM((DIM,), jnp.float32),
            pltpu.VMEM((16,), jnp.float32),
            pltpu.SemaphoreType.DMA,
        ],
    )
    eps, h_a = run(phi2, emb_table, bias, w1, b16)
    return eps, h_a
